# baseline (speedup 1.0000x reference)
import functools

import jax
import jax.numpy as jnp
from jax import lax
from jax.experimental import pallas as pl
from jax.experimental.pallas import tpu as pltpu
from jax.experimental.pallas import tpu_sc as plsc

_NC = 2
_NS = 16
_NW = _NC * _NS
_CH = 400
_NB = 2


def _cdiv(a, b):
    return (a + b - 1) // b


def _chunk(ref, c):
    return ref.at[pl.ds(c * _CH, _CH), :]


def _copy_table(src, dst, nch, w, bufs, ld_sems, st_sems):
    n_outer = _cdiv(_cdiv(nch, _NW), _NB)

    for b in range(_NB):
        c = b * _NW + w

        @pl.when(c < nch)
        def _():
            pltpu.make_async_copy(_chunk(src, c), bufs[b], ld_sems[b]).start()

    def outer(io, carry):
        for b in range(_NB):
            j = io * _NB + b
            c = j * _NW + w
            c_nxt = c + _NB * _NW

            @pl.when(c < nch)
            def _():
                pltpu.make_async_copy(_chunk(src, c), bufs[b], ld_sems[b]).wait()
                pltpu.make_async_copy(bufs[b], _chunk(dst, c), st_sems[b]).start()

            @pl.when(c_nxt < nch)
            def _():
                pltpu.make_async_copy(bufs[b], _chunk(dst, c), st_sems[b]).wait()
                pltpu.make_async_copy(_chunk(src, c_nxt), bufs[b], ld_sems[b]).start()

        return carry

    lax.fori_loop(0, n_outer, outer, 0)

    for b in range(_NB):
        c0 = b * _NW + w

        @pl.when(c0 < nch)
        def _():
            pltpu.make_async_copy(bufs[b], _chunk(dst, c0), st_sems[b]).wait()


def _sc_copy_body(u_hbm, i_hbm, u_out, i_out, buf0, buf1, ls0, ls1, ss0, ss1):
    w = lax.axis_index("s") * _NC + lax.axis_index("c")
    bufs = (buf0, buf1)
    ld_sems = (ls0, ls1)
    st_sems = (ss0, ss1)
    _copy_table(i_hbm, i_out, i_hbm.shape[0] // _CH, w, bufs, ld_sems, st_sems)
    _copy_table(u_hbm, u_out, u_hbm.shape[0] // _CH, w, bufs, ld_sems, st_sems)


def kernel(embedding_user, embedding_item):
    u_shape, i_shape = embedding_user.shape, embedding_item.shape
    u2 = embedding_user.reshape(-1, 128)
    i2 = embedding_item.reshape(-1, 128)

    mesh = plsc.VectorSubcoreMesh(core_axis_name="c", subcore_axis_name="s")
    sc_copy = functools.partial(
        pl.kernel,
        out_type=[
            jax.ShapeDtypeStruct(u2.shape, u2.dtype),
            jax.ShapeDtypeStruct(i2.shape, i2.dtype),
        ],
        mesh=mesh,
        scratch_types=[
            pltpu.VMEM((_CH, 128), jnp.float32),
            pltpu.VMEM((_CH, 128), jnp.float32),
            pltpu.SemaphoreType.DMA,
            pltpu.SemaphoreType.DMA,
            pltpu.SemaphoreType.DMA,
            pltpu.SemaphoreType.DMA,
        ],
    )(_sc_copy_body)

    u_out, i_out = sc_copy(u2, i2)
    return (u_out.reshape(u_shape), i_out.reshape(i_shape))


# speedup vs baseline: 5.5166x; 5.5166x over previous
import jax
import jax.numpy as jnp
from jax.experimental import pallas as pl
from jax.experimental.pallas import tpu as pltpu


def _copy_block(x_ref, o_ref):
    o_ref[...] = x_ref[...]


def _pipelined_copy(x, block_cols):
    rows, cols = x.shape
    return pl.pallas_call(
        _copy_block,
        grid=(pl.cdiv(cols, block_cols),),
        in_specs=[pl.BlockSpec((rows, block_cols), lambda i: (0, i))],
        out_specs=pl.BlockSpec((rows, block_cols), lambda i: (0, i)),
        out_shape=jax.ShapeDtypeStruct(x.shape, x.dtype),
    )(x)


def kernel(embedding_user, embedding_item):
    u_t = embedding_user.T
    i_t = embedding_item.T
    u_out = _pipelined_copy(u_t, 4096)
    i_out = _pipelined_copy(i_t, 4096)
    return (u_out.T, i_out.T)


# speedup vs baseline: 8.3732x; 1.5178x over previous
import jax
import jax.numpy as jnp
from jax.experimental import pallas as pl
from jax.experimental.pallas import tpu as pltpu


def _copy_block(x_ref, o_ref):
    o_ref[...] = x_ref[...]


def _pipelined_copy(x, block_cols):
    rows, cols = x.shape
    return pl.pallas_call(
        _copy_block,
        grid=(pl.cdiv(cols, block_cols),),
        in_specs=[pl.BlockSpec((rows, block_cols), lambda i: (0, i))],
        out_specs=pl.BlockSpec((rows, block_cols), lambda i: (0, i)),
        out_shape=jax.ShapeDtypeStruct(x.shape, x.dtype),
    )(x)


def kernel(embedding_user, embedding_item):
    u_t = embedding_user.T
    i_t = embedding_item.T
    u_out = _pipelined_copy(u_t, 16384)
    i_out = _pipelined_copy(i_t, 16384)
    return (u_out.T, i_out.T)


# speedup vs baseline: 8.4802x; 1.0128x over previous
import jax
import jax.numpy as jnp
from jax.experimental import pallas as pl
from jax.experimental.pallas import tpu as pltpu


def _copy_block(x_ref, o_ref):
    o_ref[...] = x_ref[...]


def _pipelined_copy(x, block_cols):
    rows, cols = x.shape
    return pl.pallas_call(
        _copy_block,
        grid=(pl.cdiv(cols, block_cols),),
        in_specs=[pl.BlockSpec((rows, block_cols), lambda i: (0, i))],
        out_specs=pl.BlockSpec((rows, block_cols), lambda i: (0, i)),
        out_shape=jax.ShapeDtypeStruct(x.shape, x.dtype),
    )(x)


def kernel(embedding_user, embedding_item):
    u_t = embedding_user.T
    i_t = embedding_item.T
    u_out = _pipelined_copy(u_t, 8192)
    i_out = _pipelined_copy(i_t, 32768)
    return (u_out.T, i_out.T)
